# Initial kernel scaffold; baseline (speedup 1.0000x reference)
import functools

import jax
import jax.numpy as jnp
from jax import lax
from jax.experimental import pallas as pl
from jax.experimental.pallas import tpu as pltpu
from jax.experimental.pallas import tpu_sc as plsc

D_MODEL = 128


def _tab_body(wc_ref, wo_ref, we_ref, weo_ref, tc_ref, te_ref):
    tc_ref[...] = wc_ref[...][:, None, :] + wo_ref[...][None, :, :]
    te_ref[...] = we_ref[...][:, None, :] + weo_ref[...][None, :, :]


def _build_tab(W_cperm, W_cori, W_eperm, W_eori):
    tab_c, tab_e = pl.pallas_call(
        _tab_body,
        out_shape=[
            jax.ShapeDtypeStruct((8, 3, D_MODEL), jnp.float32),
            jax.ShapeDtypeStruct((12, 2, D_MODEL), jnp.float32),
        ],
    )(W_cperm, W_cori, W_eperm, W_eori)
    return jnp.concatenate(
        [tab_c.reshape(24, D_MODEL), tab_e.reshape(24, D_MODEL)], axis=0
    )


def kernel(tokens, W_cperm, W_cori, W_eperm, W_eori):
    tokens = tokens.astype(jnp.int32)
    B, T = tokens.shape
    J = 20

    tab = _build_tab(W_cperm, W_cori, W_eperm, W_eori)
    tok_flat = tokens.reshape(-1)

    info = plsc.get_sparse_core_info()
    NC, NS = info.num_cores, info.num_subcores
    NW = NC * NS
    rows_w = B // NW
    out_rows_w = rows_w * J
    n_chunks = out_rows_w // 128

    mesh = plsc.VectorSubcoreMesh(core_axis_name="c", subcore_axis_name="s")

    @functools.partial(
        pl.kernel,
        out_type=jax.ShapeDtypeStruct((B * J, D_MODEL), jnp.float32),
        mesh=mesh,
        scratch_types=[
            pltpu.VMEM((rows_w * T,), jnp.int32),
            pltpu.VMEM((n_chunks, 128), jnp.int32),
            pltpu.VMEM((128, D_MODEL), jnp.float32),
            pltpu.SemaphoreType.DMA,
        ],
    )
    def run(tok_hbm, tab_hbm, out_hbm, tok_v, idx_v, rows_v, sem):
        wid = lax.axis_index("s") * NC + lax.axis_index("c")
        pltpu.sync_copy(tok_hbm.at[pl.ds(wid * (rows_w * T), rows_w * T)], tok_v)

        lanes = lax.iota(jnp.int32, 16)

        def idx_body(g, carry):
            r = g * 16 + lanes
            rb = r * T
            pr = r * J
            for j in range(8):
                a = plsc.load_gather(tok_v, [rb + j])
                b = plsc.load_gather(tok_v, [rb + (j + 8)])
                p = pr + j
                plsc.store_scatter(idx_v, [p >> 7, p & 127], a * 3 + b)
            for j in range(12):
                a = plsc.load_gather(tok_v, [rb + (16 + j)])
                b = plsc.load_gather(tok_v, [rb + (28 + j)])
                p = pr + (8 + j)
                plsc.store_scatter(idx_v, [p >> 7, p & 127], a * 2 + b + 24)
            return carry

        lax.fori_loop(0, rows_w // 16, idx_body, 0)

        out_base = wid * out_rows_w

        def gather_body(c, carry):
            pltpu.async_copy(tab_hbm.at[idx_v.at[c]], rows_v, sem).wait()
            pltpu.sync_copy(rows_v, out_hbm.at[pl.ds(out_base + c * 128, 128)])
            return carry

        lax.fori_loop(0, n_chunks, gather_body, 0)

    out = run(tok_flat, tab)
    return out.reshape(B, J, D_MODEL)


# speedup vs baseline: 1.1552x; 1.1552x over previous
import functools

import jax
import jax.numpy as jnp
from jax import lax
from jax.experimental import pallas as pl
from jax.experimental.pallas import tpu as pltpu
from jax.experimental.pallas import tpu_sc as plsc

D_MODEL = 128


def _tab_body(wc_ref, wo_ref, we_ref, weo_ref, tc_ref, te_ref):
    tc_ref[...] = wc_ref[...][:, None, :] + wo_ref[...][None, :, :]
    te_ref[...] = we_ref[...][:, None, :] + weo_ref[...][None, :, :]


def _build_tab(W_cperm, W_cori, W_eperm, W_eori):
    tab_c, tab_e = pl.pallas_call(
        _tab_body,
        out_shape=[
            jax.ShapeDtypeStruct((8, 3, D_MODEL), jnp.float32),
            jax.ShapeDtypeStruct((12, 2, D_MODEL), jnp.float32),
        ],
    )(W_cperm, W_cori, W_eperm, W_eori)
    return jnp.concatenate(
        [tab_c.reshape(24, D_MODEL), tab_e.reshape(24, D_MODEL)], axis=0
    )


def kernel(tokens, W_cperm, W_cori, W_eperm, W_eori):
    tokens = tokens.astype(jnp.int32)
    B, T = tokens.shape
    J = 20

    tab = _build_tab(W_cperm, W_cori, W_eperm, W_eori)
    tok_flat = tokens.reshape(-1)

    info = plsc.get_sparse_core_info()
    NC, NS = info.num_cores, info.num_subcores
    NW = NC * NS
    rows_w = B // NW
    out_rows_w = rows_w * J
    n_chunks = out_rows_w // 128

    mesh = plsc.VectorSubcoreMesh(core_axis_name="c", subcore_axis_name="s")

    @functools.partial(
        pl.kernel,
        out_type=jax.ShapeDtypeStruct((B * J, D_MODEL), jnp.float32),
        mesh=mesh,
        compiler_params=pltpu.CompilerParams(
            use_tc_tiling_on_sc=False, needs_layout_passes=False
        ),
        scratch_types=[
            pltpu.VMEM((rows_w * T,), jnp.int32),
            pltpu.VMEM((n_chunks, 128), jnp.int32),
            pltpu.VMEM((128, D_MODEL), jnp.float32),
            pltpu.SemaphoreType.DMA,
        ],
    )
    def run(tok_hbm, tab_hbm, out_hbm, tok_v, idx_v, rows_v, sem):
        wid = lax.axis_index("s") * NC + lax.axis_index("c")
        pltpu.sync_copy(tok_hbm.at[pl.ds(wid * (rows_w * T), rows_w * T)], tok_v)

        lanes = lax.iota(jnp.int32, 16)

        def idx_body(g, carry):
            r = g * 16 + lanes
            rb = r * T
            pr = r * J
            for j in range(8):
                a = plsc.load_gather(tok_v, [rb + j])
                b = plsc.load_gather(tok_v, [rb + (j + 8)])
                p = pr + j
                plsc.store_scatter(idx_v, [p >> 7, p & 127], a * 3 + b)
            for j in range(12):
                a = plsc.load_gather(tok_v, [rb + (16 + j)])
                b = plsc.load_gather(tok_v, [rb + (28 + j)])
                p = pr + (8 + j)
                plsc.store_scatter(idx_v, [p >> 7, p & 127], a * 2 + b + 24)
            return carry

        lax.fori_loop(0, rows_w // 16, idx_body, 0)

        out_base = wid * out_rows_w

        def gather_body(c, carry):
            pltpu.async_copy(tab_hbm.at[idx_v.at[c]], rows_v, sem).wait()
            pltpu.sync_copy(rows_v, out_hbm.at[pl.ds(out_base + c * 128, 128)])
            return carry

        lax.fori_loop(0, n_chunks, gather_body, 0)

    out = run(tok_flat, tab)
    return out.reshape(B, J, D_MODEL)
